# baseline (speedup 1.0000x reference)
import functools
import jax
import jax.numpy as jnp
from jax import lax
from jax.experimental import pallas as pl
from jax.experimental.pallas import tpu as pltpu
from jax.experimental.pallas import tpu_sc as plsc

B, T, S, V, DIM = 1024, 200, 50, 1000, 16
TP = 208
NG = 2
NGRP = B // 16
BB = 8


def _sc_body(encp_h, w_h, e_h, prev_h, enc16_v, tbl_v, e16_v,
             prev16a_v, prev16b_v, gsem, wsem):
    wid = lax.axis_index("s") * 2 + lax.axis_index("c")
    iota = lax.iota(jnp.int32, 16)
    zeros16i = jnp.zeros((16,), jnp.int32)

    def clr(r, carry):
        tbl_v[pl.ds(r * 16, 16)] = zeros16i
        return carry
    lax.fori_loop(0, V, clr, 0)

    lanes_base = iota * TP
    prev_bufs = (prev16a_v, prev16b_v)

    def fire_gathers():
        return [pltpu.async_copy(
            w_h.at[enc16_v.at[pl.ds(k * 128, 128)]],
            e16_v.at[pl.ds(k * 128, 128)], gsem)
            for k in range(16 * TP // 128)]

    def run_scan(g, prev_v):
        tag0 = (g + 1) * 256

        def aloop(t, carry):
            v = plsc.load_gather(enc16_v, [lanes_base + t])
            fidx = v * 16 + iota
            lp = plsc.load_gather(tbl_v, [fidx])
            plsc.store_scatter(tbl_v, [fidx],
                               jnp.full((16,), tag0 + t, jnp.int32))
            valid = lax.shift_right_logical(lp, 8) == (g + 1)
            prevt = jnp.where(valid, lp & 255, jnp.full((16,), -1, jnp.int32))
            plsc.store_scatter(prev_v, [lanes_base + t], prevt)
            return carry

        lax.fori_loop(0, T, aloop, 0, unroll=4)

    G0 = wid * NG
    pltpu.sync_copy(encp_h.at[G0], enc16_v)
    copies = fire_gathers()
    run_scan(0, prev_bufs[0])
    for c in copies:
        c.wait()
    wb_e = pltpu.async_copy(e16_v, e_h.at[G0], wsem)
    wb_p = pltpu.async_copy(prev_bufs[0], prev_h.at[G0], wsem)

    G1 = G0 + 1
    pltpu.sync_copy(encp_h.at[G1], enc16_v)
    run_scan(1, prev_bufs[1])
    wb_e.wait()
    copies = fire_gathers()
    for c in copies:
        c.wait()
    wb_p.wait()
    pltpu.sync_copy(e16_v, e_h.at[G1])
    pltpu.sync_copy(prev_bufs[1], prev_h.at[G1])


def _tc_kernel(lo_ref, hi_ref, prev_ref, e_ref, bias_ref, out_ref):
    lo = lo_ref[...]
    hi = hi_ref[...]
    prev = prev_ref[...]
    pos = lax.broadcasted_iota(jnp.int32, (BB, S, TP), 2)
    lob = lo[:, :, None]
    mask = ((pos >= lob) & (pos < hi[:, :, None])
            & (prev[:, None, :] < lob)).astype(jnp.float32)
    out = lax.dot_general(
        mask, e_ref[...],
        dimension_numbers=(((2,), (1,)), ((0,), (0,))),
        preferred_element_type=jnp.float32,
    )
    out_ref[...] = out + bias_ref[...][None, None, :]


def kernel(word_encs, span_idxs, W, bias):
    enc = word_encs.astype(jnp.int32)
    enc_pad = jnp.zeros((B, TP), jnp.int32).at[:, :T].set(enc)
    encp_h = enc_pad.reshape(NGRP, 16 * TP)

    mesh = plsc.VectorSubcoreMesh(core_axis_name="c", subcore_axis_name="s")
    sc = functools.partial(
        pl.kernel,
        out_type=(
            jax.ShapeDtypeStruct((NGRP, 16 * TP, DIM), jnp.float32),
            jax.ShapeDtypeStruct((NGRP, 16 * TP), jnp.int32),
        ),
        mesh=mesh,
        compiler_params=pltpu.CompilerParams(
            needs_layout_passes=False, use_tc_tiling_on_sc=False,
            skip_device_barrier=True),
        scratch_types=[
            pltpu.VMEM((16 * TP,), jnp.int32),
            pltpu.VMEM((V * 16,), jnp.int32),
            pltpu.VMEM((16 * TP, DIM), jnp.float32),
            pltpu.VMEM((16 * TP,), jnp.int32),
            pltpu.VMEM((16 * TP,), jnp.int32),
            pltpu.SemaphoreType.DMA,
            pltpu.SemaphoreType.DMA,
        ],
    )(_sc_body)
    e_rows, prev = sc(encp_h, W.astype(jnp.float32))
    e_rows = e_rows.reshape(B, TP, DIM)
    prev = prev.reshape(B, TP)

    lo = span_idxs[:, :, 0].astype(jnp.int32)
    hi = span_idxs[:, :, 1].astype(jnp.int32)
    return pl.pallas_call(
        _tc_kernel,
        grid=(B // BB,),
        in_specs=[
            pl.BlockSpec((BB, S), lambda g: (g, 0)),
            pl.BlockSpec((BB, S), lambda g: (g, 0)),
            pl.BlockSpec((BB, TP), lambda g: (g, 0)),
            pl.BlockSpec((BB, TP, DIM), lambda g: (g, 0, 0)),
            pl.BlockSpec((DIM,), lambda g: (0,)),
        ],
        out_specs=pl.BlockSpec((BB, S, DIM), lambda g: (g, 0, 0)),
        out_shape=jax.ShapeDtypeStruct((B, S, DIM), jnp.float32),
    )(lo, hi, prev, e_rows, bias.astype(jnp.float32))


# speedup vs baseline: 1.0009x; 1.0009x over previous
import functools
import jax
import jax.numpy as jnp
from jax import lax
from jax.experimental import pallas as pl
from jax.experimental.pallas import tpu as pltpu
from jax.experimental.pallas import tpu_sc as plsc

B, T, S, V, DIM = 1024, 200, 50, 1000, 16
TP = 208
NG = 2
NGRP = B // 16
BB = 8


def _sc_body(encp_h, w_h, e_h, prev_h, enc16_v, tbl_v, e16_v,
             prev16a_v, prev16b_v, gsem, wsem):
    wid = lax.axis_index("s") * 2 + lax.axis_index("c")
    iota = lax.iota(jnp.int32, 16)
    zeros16i = jnp.zeros((16,), jnp.int32)

    def clr(r, carry):
        tbl_v[pl.ds(r * 16, 16)] = zeros16i
        return carry
    lax.fori_loop(0, V, clr, 0)

    lanes_base = iota * TP
    prev_bufs = (prev16a_v, prev16b_v)

    def fire_gathers():
        return [pltpu.async_copy(
            w_h.at[enc16_v.at[pl.ds(l0 * TP + cc * 104, 104)]],
            e16_v.at[l0, pl.ds(cc * 104, 104)], gsem)
            for l0 in range(16) for cc in range(2)]

    def run_scan(g, prev_v):
        tag0 = (g + 1) * 256

        def aloop(t, carry):
            v = plsc.load_gather(enc16_v, [lanes_base + t])
            fidx = v * 16 + iota
            lp = plsc.load_gather(tbl_v, [fidx])
            plsc.store_scatter(tbl_v, [fidx],
                               jnp.full((16,), tag0 + t, jnp.int32))
            valid = lax.shift_right_logical(lp, 8) == (g + 1)
            prevt = jnp.where(valid, lp & 255, jnp.full((16,), -1, jnp.int32))
            plsc.store_scatter(prev_v, [iota, jnp.full((16,), t, jnp.int32)],
                               prevt)
            return carry

        lax.fori_loop(0, T, aloop, 0, unroll=4)

    G0 = wid * NG
    pltpu.sync_copy(encp_h.at[G0], enc16_v)
    copies = fire_gathers()
    run_scan(0, prev_bufs[0])
    for c in copies:
        c.wait()
    wb_e = pltpu.async_copy(e16_v, e_h.at[pl.ds(G0 * 16, 16)], wsem)
    wb_p = pltpu.async_copy(prev_bufs[0], prev_h.at[pl.ds(G0 * 16, 16)], wsem)

    G1 = G0 + 1
    pltpu.sync_copy(encp_h.at[G1], enc16_v)
    run_scan(1, prev_bufs[1])
    wb_e.wait()
    copies = fire_gathers()
    for c in copies:
        c.wait()
    wb_p.wait()
    pltpu.sync_copy(e16_v, e_h.at[pl.ds(G1 * 16, 16)])
    pltpu.sync_copy(prev_bufs[1], prev_h.at[pl.ds(G1 * 16, 16)])


def _tc_kernel(lo_ref, hi_ref, prev_ref, e_ref, bias_ref, out_ref):
    lo = lo_ref[...]
    hi = hi_ref[...]
    prev = prev_ref[...]
    pos = lax.broadcasted_iota(jnp.int32, (BB, S, TP), 2)
    lob = lo[:, :, None]
    mask = ((pos >= lob) & (pos < hi[:, :, None])
            & (prev[:, None, :] < lob)).astype(jnp.float32)
    out = lax.dot_general(
        mask, e_ref[...],
        dimension_numbers=(((2,), (1,)), ((0,), (0,))),
        preferred_element_type=jnp.float32,
    )
    out_ref[...] = out + bias_ref[...][None, None, :]


def kernel(word_encs, span_idxs, W, bias):
    enc = word_encs.astype(jnp.int32)
    enc_pad = jnp.zeros((B, TP), jnp.int32).at[:, :T].set(enc)
    encp_h = enc_pad.reshape(NGRP, 16 * TP)

    mesh = plsc.VectorSubcoreMesh(core_axis_name="c", subcore_axis_name="s")
    sc = functools.partial(
        pl.kernel,
        out_type=(
            jax.ShapeDtypeStruct((B, TP, DIM), jnp.float32),
            jax.ShapeDtypeStruct((B, TP), jnp.int32),
        ),
        mesh=mesh,
        compiler_params=pltpu.CompilerParams(
            needs_layout_passes=False, use_tc_tiling_on_sc=False,
            skip_device_barrier=True),
        scratch_types=[
            pltpu.VMEM((16 * TP,), jnp.int32),
            pltpu.VMEM((V * 16,), jnp.int32),
            pltpu.VMEM((16, TP, DIM), jnp.float32),
            pltpu.VMEM((16, TP), jnp.int32),
            pltpu.VMEM((16, TP), jnp.int32),
            pltpu.SemaphoreType.DMA,
            pltpu.SemaphoreType.DMA,
        ],
    )(_sc_body)
    e_rows, prev = sc(encp_h, W.astype(jnp.float32))

    lo = span_idxs[:, :, 0].astype(jnp.int32)
    hi = span_idxs[:, :, 1].astype(jnp.int32)
    return pl.pallas_call(
        _tc_kernel,
        grid=(B // BB,),
        in_specs=[
            pl.BlockSpec((BB, S), lambda g: (g, 0)),
            pl.BlockSpec((BB, S), lambda g: (g, 0)),
            pl.BlockSpec((BB, TP), lambda g: (g, 0)),
            pl.BlockSpec((BB, TP, DIM), lambda g: (g, 0, 0)),
            pl.BlockSpec((DIM,), lambda g: (0,)),
        ],
        out_specs=pl.BlockSpec((BB, S, DIM), lambda g: (g, 0, 0)),
        out_shape=jax.ShapeDtypeStruct((B, S, DIM), jnp.float32),
    )(lo, hi, prev, e_rows, bias.astype(jnp.float32))
